# Initial kernel scaffold; baseline (speedup 1.0000x reference)
import jax
import jax.numpy as jnp
from jax.experimental import pallas as pl


def kernel(features, ec_W0, ec_b0, ec_W1, ec_b1, ec_W2, ec_b2, fc_W0, fc_b0, fc_W1, fc_b1, gcn_W0, gcn_b0, gcn_W1, gcn_b1, gcn_W2, gcn_b2):
    raise NotImplementedError("write your pallas kernel here")


# speedup vs baseline: 34.9896x; 34.9896x over previous
import jax
import jax.numpy as jnp
from jax.experimental import pallas as pl
from jax.experimental.pallas import tpu as pltpu

_BI = 32


def _fused(x_ref, w0_ref, b0_ref, w1_ref, b1_ref, w2t_ref, b2_ref,
           fw0_ref, fb0_ref, fw1_ref, fb1_ref,
           gw0_ref, gb0_ref, gw1_ref, gb1_ref, gw2_ref, gb2_ref,
           out_ref, a_ref, pd_ref):
    x = x_ref[:, :]
    n = x.shape[0]
    h = n // 2
    fdim = x.shape[1]
    w0 = w0_ref[:, :]
    p = x @ w0[:fdim, :] + b0_ref[:, :]
    pd_ref[:, :] = jnp.concatenate([p, p], axis=1)
    q = x @ w0[fdim:, :]
    q2 = jnp.concatenate([q[:h, :], q[h:, :]], axis=1)

    w1 = w1_ref[:, :]
    z1 = jnp.zeros_like(w1)
    w1dd = jnp.concatenate(
        [jnp.concatenate([w1, z1], axis=1),
         jnp.concatenate([z1, w1], axis=1)], axis=0)
    b1 = b1_ref[:, :]
    b1d = jnp.concatenate([b1, b1], axis=1)
    w2r = w2t_ref[:, :][None]
    b2 = b2_ref[0, 0]

    def body(i, carry):
        pi = pd_ref[pl.ds(i * _BI, _BI), :]
        t0 = jnp.maximum(pi[:, None, :] + q2[None, :, :], 0.0)
        h1 = jnp.maximum(
            t0.reshape(_BI * h, 128) @ w1dd + b1d, 0.0)
        h1r = h1.reshape(_BI, h, 64)
        e0 = jnp.sum(h1r[:, :, :32] * w2r, axis=-1)
        e1 = jnp.sum(h1r[:, :, 32:] * w2r, axis=-1)
        e = jnp.concatenate([e0, e1], axis=-1)
        a_ref[pl.ds(i * _BI, _BI), :] = jax.nn.sigmoid(e + b2)
        return carry

    jax.lax.fori_loop(0, n // _BI, body, 0)

    a = a_ref[:, :]
    f = jnp.maximum(x @ fw0_ref[:, :] + fb0_ref[:, :], 0.0)
    f = f @ fw1_ref[:, :] + fb1_ref[:, :]
    g = jnp.maximum(a @ (f @ gw0_ref[:, :]) + gb0_ref[:, :], 0.0)
    g = jnp.maximum(a @ (g @ gw1_ref[:, :]) + gb1_ref[:, :], 0.0)
    w2m = jnp.mean(gw2_ref[:, :], axis=1, keepdims=True)
    out_ref[:, :] = a @ (g @ w2m) + jnp.mean(gb2_ref[:, :])


def kernel(features, ec_W0, ec_b0, ec_W1, ec_b1, ec_W2, ec_b2,
           fc_W0, fc_b0, fc_W1, fc_b1,
           gcn_W0, gcn_b0, gcn_W1, gcn_b1, gcn_W2, gcn_b2):
    x = jnp.squeeze(features)
    n = x.shape[0]
    out = pl.pallas_call(
        _fused,
        out_shape=jax.ShapeDtypeStruct((n, 1), jnp.float32),
        scratch_shapes=[pltpu.VMEM((n, n), jnp.float32),
                        pltpu.VMEM((n, 128), jnp.float32)],
    )(x, ec_W0, ec_b0.reshape(1, -1), ec_W1, ec_b1.reshape(1, -1),
      ec_W2.reshape(1, -1), ec_b2.reshape(1, 1),
      fc_W0, fc_b0.reshape(1, -1), fc_W1, fc_b1.reshape(1, -1),
      gcn_W0, gcn_b0.reshape(1, -1), gcn_W1, gcn_b1.reshape(1, -1),
      gcn_W2, gcn_b2.reshape(1, -1))
    return out.reshape(n)


# speedup vs baseline: 93.9675x; 2.6856x over previous
import jax
import jax.numpy as jnp
from jax.experimental import pallas as pl
from jax.experimental.pallas import tpu as pltpu

_BI = 64


def _fused(x_ref, w0_ref, b0_ref, w1_ref, b1_ref, w2t_ref, b2_ref,
           fw0_ref, fb0_ref, fw1_ref, fb1_ref,
           gw0_ref, gb0_ref, gw1_ref, gb1_ref, gw2_ref, gb2_ref,
           out_ref, a_ref, pd_ref):
    x = x_ref[:, :]
    n = x.shape[0]
    h = n // 2
    fdim = x.shape[1]
    w0 = w0_ref[:, :]
    p = x @ w0[:fdim, :] + b0_ref[:, :]
    pd_ref[:, :] = jnp.concatenate([p, p], axis=1).astype(jnp.bfloat16)
    q = x @ w0[fdim:, :]
    q2 = jnp.concatenate([q[:h, :], q[h:, :]], axis=1).astype(jnp.bfloat16)

    w1 = w1_ref[:, :]
    z1 = jnp.zeros_like(w1)
    w1dd = jnp.concatenate(
        [jnp.concatenate([w1, z1], axis=1),
         jnp.concatenate([z1, w1], axis=1)], axis=0).astype(jnp.bfloat16)
    b1 = b1_ref[:, :]
    b1d = jnp.concatenate([b1, b1], axis=1)
    w2t = w2t_ref[:, :]
    zw2 = jnp.zeros_like(w2t)
    w2dd = jnp.concatenate(
        [jnp.concatenate([w2t, zw2], axis=1),
         jnp.concatenate([zw2, w2t], axis=1)], axis=0).T
    b2 = b2_ref[0, 0]

    def body(i, carry):
        pi = pd_ref[pl.ds(i * _BI, _BI), :]
        zero = jnp.zeros((), jnp.bfloat16)
        t0 = jnp.maximum(pi[:, None, :] + q2[None, :, :], zero)
        h1 = jnp.maximum(
            jax.lax.dot(t0.reshape(_BI * h, 128), w1dd,
                        preferred_element_type=jnp.float32) + b1d,
            0.0)
        ep = jax.lax.dot(h1, w2dd, preferred_element_type=jnp.float32)
        et = jnp.swapaxes(ep.reshape(_BI, h, 2), 1, 2)
        e = jnp.concatenate([et[:, 0, :], et[:, 1, :]], axis=-1)
        a_ref[pl.ds(i * _BI, _BI), :] = jax.nn.sigmoid(e + b2)
        return carry

    jax.lax.fori_loop(0, n // _BI, body, 0)

    a = a_ref[:, :]
    f = jnp.maximum(x @ fw0_ref[:, :] + fb0_ref[:, :], 0.0)
    f = f @ fw1_ref[:, :] + fb1_ref[:, :]
    g = jnp.maximum(a @ (f @ gw0_ref[:, :]) + gb0_ref[:, :], 0.0)
    g = jnp.maximum(a @ (g @ gw1_ref[:, :]) + gb1_ref[:, :], 0.0)
    w2m = jnp.mean(gw2_ref[:, :], axis=1, keepdims=True)
    out_ref[:, :] = a @ (g @ w2m) + jnp.mean(gb2_ref[:, :])


def kernel(features, ec_W0, ec_b0, ec_W1, ec_b1, ec_W2, ec_b2,
           fc_W0, fc_b0, fc_W1, fc_b1,
           gcn_W0, gcn_b0, gcn_W1, gcn_b1, gcn_W2, gcn_b2):
    x = jnp.squeeze(features)
    n = x.shape[0]
    out = pl.pallas_call(
        _fused,
        out_shape=jax.ShapeDtypeStruct((n, 1), jnp.float32),
        scratch_shapes=[pltpu.VMEM((n, n), jnp.float32),
                        pltpu.VMEM((n, 128), jnp.bfloat16)],
    )(x, ec_W0, ec_b0.reshape(1, -1), ec_W1, ec_b1.reshape(1, -1),
      ec_W2.reshape(1, -1), ec_b2.reshape(1, 1),
      fc_W0, fc_b0.reshape(1, -1), fc_W1, fc_b1.reshape(1, -1),
      gcn_W0, gcn_b0.reshape(1, -1), gcn_W1, gcn_b1.reshape(1, -1),
      gcn_W2, gcn_b2.reshape(1, -1))
    return out.reshape(n)


# speedup vs baseline: 96.7354x; 1.0295x over previous
import jax
import jax.numpy as jnp
from jax.experimental import pallas as pl
from jax.experimental.pallas import tpu as pltpu

_BI = 128


def _fused(x_ref, w0_ref, b0_ref, w1_ref, b1_ref, w2t_ref, b2_ref,
           fw0_ref, fb0_ref, fw1_ref, fb1_ref,
           gw0_ref, gb0_ref, gw1_ref, gb1_ref, gw2_ref, gb2_ref,
           out_ref, a_ref, pd_ref):
    x = x_ref[:, :]
    n = x.shape[0]
    h = n // 2
    fdim = x.shape[1]
    w0 = w0_ref[:, :]
    p = x @ w0[:fdim, :] + b0_ref[:, :]
    pd_ref[:, :] = jnp.concatenate([p, p], axis=1).astype(jnp.bfloat16)
    q = x @ w0[fdim:, :]
    q2 = jnp.concatenate([q[:h, :], q[h:, :]], axis=1).astype(jnp.bfloat16)

    w1 = w1_ref[:, :]
    z1 = jnp.zeros_like(w1)
    w1dd = jnp.concatenate(
        [jnp.concatenate([w1, z1], axis=1),
         jnp.concatenate([z1, w1], axis=1)], axis=0).astype(jnp.bfloat16)
    b1 = b1_ref[:, :]
    b1d = jnp.concatenate([b1, b1], axis=1)
    w2t = w2t_ref[:, :]
    zw2 = jnp.zeros_like(w2t)
    w2dd = jnp.concatenate(
        [jnp.concatenate([w2t, zw2], axis=1),
         jnp.concatenate([zw2, w2t], axis=1)], axis=0).T
    b2 = b2_ref[0, 0]

    def body(i, carry):
        pi = pd_ref[pl.ds(i * _BI, _BI), :]
        zero = jnp.zeros((), jnp.bfloat16)
        t0 = jnp.maximum(pi[:, None, :] + q2[None, :, :], zero)
        h1 = jnp.maximum(
            jax.lax.dot(t0.reshape(_BI * h, 128), w1dd,
                        preferred_element_type=jnp.float32) + b1d,
            0.0)
        ep = jax.lax.dot(h1, w2dd, preferred_element_type=jnp.float32)
        et = jnp.swapaxes(ep.reshape(_BI, h, 2), 1, 2)
        e = jnp.concatenate([et[:, 0, :], et[:, 1, :]], axis=-1)
        a_ref[pl.ds(i * _BI, _BI), :] = jax.nn.sigmoid(e + b2)
        return carry

    jax.lax.fori_loop(0, n // _BI, body, 0)

    a = a_ref[:, :]
    f = jnp.maximum(x @ fw0_ref[:, :] + fb0_ref[:, :], 0.0)
    f = f @ fw1_ref[:, :] + fb1_ref[:, :]
    g = jnp.maximum(a @ (f @ gw0_ref[:, :]) + gb0_ref[:, :], 0.0)
    g = jnp.maximum(a @ (g @ gw1_ref[:, :]) + gb1_ref[:, :], 0.0)
    w2m = jnp.mean(gw2_ref[:, :], axis=1, keepdims=True)
    out_ref[:, :] = a @ (g @ w2m) + jnp.mean(gb2_ref[:, :])


def kernel(features, ec_W0, ec_b0, ec_W1, ec_b1, ec_W2, ec_b2,
           fc_W0, fc_b0, fc_W1, fc_b1,
           gcn_W0, gcn_b0, gcn_W1, gcn_b1, gcn_W2, gcn_b2):
    x = jnp.squeeze(features)
    n = x.shape[0]
    out = pl.pallas_call(
        _fused,
        out_shape=jax.ShapeDtypeStruct((n, 1), jnp.float32),
        scratch_shapes=[pltpu.VMEM((n, n), jnp.float32),
                        pltpu.VMEM((n, 128), jnp.bfloat16)],
    )(x, ec_W0, ec_b0.reshape(1, -1), ec_W1, ec_b1.reshape(1, -1),
      ec_W2.reshape(1, -1), ec_b2.reshape(1, 1),
      fc_W0, fc_b0.reshape(1, -1), fc_W1, fc_b1.reshape(1, -1),
      gcn_W0, gcn_b0.reshape(1, -1), gcn_W1, gcn_b1.reshape(1, -1),
      gcn_W2, gcn_b2.reshape(1, -1))
    return out.reshape(n)


# speedup vs baseline: 107.7969x; 1.1143x over previous
import jax
import jax.numpy as jnp
from jax.experimental import pallas as pl
from jax.experimental.pallas import tpu as pltpu

_BI = 128
_G = 4


def _blockdiag(m, g):
    rows, cols = m.shape
    z = jnp.zeros_like(m)
    out_rows = []
    for r in range(g):
        out_rows.append(jnp.concatenate(
            [m if c == r else z for c in range(g)], axis=1))
    return jnp.concatenate(out_rows, axis=0)


def _fused(x_ref, w0_ref, b0_ref, w1_ref, b1_ref, w2t_ref, b2_ref,
           fw0_ref, fb0_ref, fw1_ref, fb1_ref,
           gw0_ref, gb0_ref, gw1_ref, gb1_ref, gw2_ref, gb2_ref,
           out_ref, a_ref, pd_ref):
    x = x_ref[:, :]
    n = x.shape[0]
    h = n // _G
    fdim = x.shape[1]
    w0 = w0_ref[:, :]
    p = x @ w0[:fdim, :] + b0_ref[:, :]
    pd_ref[:, :] = jnp.concatenate([p] * _G, axis=1).astype(jnp.bfloat16)
    q = x @ w0[fdim:, :]
    q2 = jnp.concatenate(
        [q[c * h:(c + 1) * h, :] for c in range(_G)],
        axis=1).astype(jnp.bfloat16)

    w1dd = _blockdiag(w1_ref[:, :], _G).astype(jnp.bfloat16)
    b1 = b1_ref[:, :]
    b1d = jnp.concatenate([b1] * _G, axis=1)
    w2t = w2t_ref[:, :]
    w2dd = _blockdiag(w2t.T, _G)
    b2 = b2_ref[0, 0]

    def body(i, carry):
        pi = pd_ref[pl.ds(i * _BI, _BI), :]
        zero = jnp.zeros((), jnp.bfloat16)
        t0 = jnp.maximum(pi[:, None, :] + q2[None, :, :], zero)
        h1 = jnp.maximum(
            jax.lax.dot(t0.reshape(_BI * h, 64 * _G), w1dd,
                        preferred_element_type=jnp.float32) + b1d,
            0.0)
        ep = jax.lax.dot(h1, w2dd, preferred_element_type=jnp.float32)
        et = jnp.swapaxes(ep.reshape(_BI, h, _G), 1, 2)
        e = jnp.concatenate([et[:, c, :] for c in range(_G)], axis=-1)
        a_ref[pl.ds(i * _BI, _BI), :] = jax.nn.sigmoid(e + b2)
        return carry

    jax.lax.fori_loop(0, n // _BI, body, 0)

    a = a_ref[:, :]
    f = jnp.maximum(x @ fw0_ref[:, :] + fb0_ref[:, :], 0.0)
    f = f @ fw1_ref[:, :] + fb1_ref[:, :]
    g = jnp.maximum(a @ (f @ gw0_ref[:, :]) + gb0_ref[:, :], 0.0)
    g = jnp.maximum(a @ (g @ gw1_ref[:, :]) + gb1_ref[:, :], 0.0)
    w2m = jnp.mean(gw2_ref[:, :], axis=1, keepdims=True)
    out_ref[:, :] = a @ (g @ w2m) + jnp.mean(gb2_ref[:, :])


def kernel(features, ec_W0, ec_b0, ec_W1, ec_b1, ec_W2, ec_b2,
           fc_W0, fc_b0, fc_W1, fc_b1,
           gcn_W0, gcn_b0, gcn_W1, gcn_b1, gcn_W2, gcn_b2):
    x = jnp.squeeze(features)
    n = x.shape[0]
    out = pl.pallas_call(
        _fused,
        out_shape=jax.ShapeDtypeStruct((n, 1), jnp.float32),
        scratch_shapes=[pltpu.VMEM((n, n), jnp.float32),
                        pltpu.VMEM((n, 64 * _G), jnp.bfloat16)],
    )(x, ec_W0, ec_b0.reshape(1, -1), ec_W1, ec_b1.reshape(1, -1),
      ec_W2.reshape(1, -1), ec_b2.reshape(1, 1),
      fc_W0, fc_b0.reshape(1, -1), fc_W1, fc_b1.reshape(1, -1),
      gcn_W0, gcn_b0.reshape(1, -1), gcn_W1, gcn_b1.reshape(1, -1),
      gcn_W2, gcn_b2.reshape(1, -1))
    return out.reshape(n)
